# Initial kernel scaffold; baseline (speedup 1.0000x reference)
import jax
import jax.numpy as jnp
from jax.experimental import pallas as pl


def kernel(x, pos, edge_index, cluster1, cluster2, cluster3, cluster4, W1, root1, b1, W2, root2, b2, W3, root3, b3, W4, root4, b4, fc1_w, fc1_b, fc2_w, fc2_b):
    raise NotImplementedError("write your pallas kernel here")


# speedup vs baseline: 1.0017x; 1.0017x over previous
import jax
import jax.numpy as jnp
import numpy as np
from jax.experimental import pallas as pl
from jax.experimental.pallas import tpu as pltpu

_KS = 5
_N1, _N2, _N3, _N4 = 2500, 625, 156, 8

_OFFS = np.array([[i, j, k] for i in (0, 1) for j in (0, 1) for k in (0, 1)],
                 dtype=np.int32)


def _pseudo(pos, ei):
    src, dst = ei[0], ei[1]
    delta = pos[dst] - pos[src]
    m = jnp.max(jnp.abs(delta))
    p = delta / (2.0 * jnp.maximum(m, 1e-8)) + 0.5
    return jnp.clip(p, 0.0, 1.0 - 1e-6)


def _sconv(x, ei, pseudo, W, root, bias):
    n = x.shape[0]
    src, dst = ei[0], ei[1]
    p = pseudo * (_KS - 1)
    lo = jnp.clip(jnp.floor(p), 0, _KS - 2).astype(jnp.int32)
    frac = p - lo.astype(p.dtype)
    off = jnp.asarray(_OFFS)
    idx3 = lo[:, None, :] + off[None, :, :]
    basis = jnp.prod(jnp.where(off[None] == 1, frac[:, None, :],
                               1.0 - frac[:, None, :]), axis=-1)
    wi = (idx3[..., 0] * _KS + idx3[..., 1]) * _KS + idx3[..., 2]
    proj = jnp.einsum('ni,kio->nko', x, W)
    gath = proj[src[:, None], wi]
    msg = jnp.sum(basis[..., None] * gath, axis=1)
    agg = jnp.zeros((n, W.shape[-1]), x.dtype).at[dst].add(msg)
    deg = jnp.zeros((n,), x.dtype).at[dst].add(1.0)
    agg = agg / jnp.clip(deg, 1.0)[:, None]
    return agg + x @ root + bias


def _pool(x, pos, cluster, num_clusters):
    nx = jax.ops.segment_max(x, cluster, num_segments=num_clusters)
    nx = jnp.where(jnp.isfinite(nx), nx, 0.0)
    cnt = jax.ops.segment_sum(jnp.ones(cluster.shape, pos.dtype), cluster,
                              num_segments=num_clusters)
    npos = jax.ops.segment_sum(pos, cluster, num_segments=num_clusters)
    npos = npos / jnp.clip(cnt, 1.0)[:, None]
    return nx, npos


def _head_body(h_ref, w1_ref, b1_ref, w2_ref, b2_ref, out_ref):
    z = jnp.dot(h_ref[...], w1_ref[...], preferred_element_type=jnp.float32)
    z = z + b1_ref[...]
    z = jnp.where(z > 0, z, jnp.exp(z) - 1.0)
    z = jnp.dot(z, w2_ref[...], preferred_element_type=jnp.float32)
    z = z + b2_ref[...]
    m = jnp.max(z, axis=-1, keepdims=True)
    s = z - m
    lse = jnp.log(jnp.sum(jnp.exp(s), axis=-1, keepdims=True))
    out_ref[...] = s - lse


def _head(h, fc1_w, fc1_b, fc2_w, fc2_b):
    return pl.pallas_call(
        _head_body,
        out_shape=jax.ShapeDtypeStruct((1, 10), jnp.float32),
    )(h, fc1_w, fc1_b.reshape(1, -1), fc2_w, fc2_b.reshape(1, -1))


def kernel(x, pos, edge_index, cluster1, cluster2, cluster3, cluster4,
           W1, root1, b1, W2, root2, b2, W3, root3, b3, W4, root4, b4,
           fc1_w, fc1_b, fc2_w, fc2_b):
    E = edge_index.shape[1]
    h = jax.nn.elu(_sconv(x, edge_index, _pseudo(pos, edge_index), W1, root1, b1))
    h, pos1 = _pool(h, pos, cluster1, _N1)
    ei2 = cluster1[edge_index][:, : E // 4]
    h = jax.nn.elu(_sconv(h, ei2, _pseudo(pos1, ei2), W2, root2, b2))
    h, pos2 = _pool(h, pos1, cluster2, _N2)
    ei3 = cluster2[ei2][:, : E // 16]
    h = jax.nn.elu(_sconv(h, ei3, _pseudo(pos2, ei3), W3, root3, b3))
    h, pos3 = _pool(h, pos2, cluster3, _N3)
    ei4 = cluster3[ei3][:, : E // 64]
    h = jax.nn.elu(_sconv(h, ei4, _pseudo(pos3, ei4), W4, root4, b4))
    h = jax.ops.segment_max(h, cluster4, num_segments=_N4)
    h = jnp.where(jnp.isfinite(h), h, 0.0)
    z = h.reshape(1, -1)
    return _head(z, fc1_w, fc1_b, fc2_w, fc2_b)


# speedup vs baseline: 2.1681x; 2.1645x over previous
import functools

import jax
import jax.numpy as jnp
import numpy as np
from jax import lax
from jax.experimental import pallas as pl
from jax.experimental.pallas import tpu as pltpu
from jax.experimental.pallas import tpu_sc as plsc

_KS = 5
_N0 = 10000
_E0 = 160000
_N1, _N2, _N3, _N4 = 2500, 625, 156, 8

_NC, _NS, _L = 2, 16, 16
_NW = _NC * _NS

_OFFS = np.array([[i, j, k] for i in (0, 1) for j in (0, 1) for k in (0, 1)],
                 dtype=np.int32)
_DOFF = [(o[0] * _KS + o[1]) * _KS + o[2] for o in _OFFS]

_f32 = jnp.float32
_i32 = jnp.int32


def _wid():
    return lax.axis_index("s") * _NC + lax.axis_index("c")


def _cid():
    return lax.axis_index("c")


def _maxk1_body(srce, dste, pos, out, src_v, dst_v, posx, v16):
    w = _wid()
    epw = _E0 // _NW
    base = w * epw
    zeros16 = jnp.zeros((_L,), _f32)
    zi16 = jnp.zeros((_L,), _i32)
    src_v[pl.ds(epw - _L, _L)] = zi16
    src_v[pl.ds(epw, _L)] = zi16
    dst_v[pl.ds(epw - _L, _L)] = zi16
    dst_v[pl.ds(epw, _L)] = zi16
    pltpu.sync_copy(srce.at[pl.ds(base, epw)], src_v.at[pl.ds(0, epw)])
    pltpu.sync_copy(dste.at[pl.ds(base, epw)], dst_v.at[pl.ds(0, epw)])
    pltpu.sync_copy(pos, posx)
    n_it = (epw + _L - 1) // _L

    def body(i, m):
        s16 = src_v[pl.ds(i * _L, _L)] * 3
        d16 = dst_v[pl.ds(i * _L, _L)] * 3
        acc = m
        for c in range(3):
            col = jnp.full((_L,), c, _i32)
            ps = plsc.load_gather(posx, [s16 + col])
            pd = plsc.load_gather(posx, [d16 + col])
            acc = jnp.maximum(acc, jnp.abs(pd - ps))
        return acc

    m = lax.fori_loop(0, n_it, body, zeros16)
    v16[...] = m
    pltpu.sync_copy(v16, out.at[w])


def _maxk1(srce, dste, pos):
    epw = _E0 // _NW
    return pl.kernel(
        _maxk1_body,
        out_type=jax.ShapeDtypeStruct((_NW, _L), _f32),
        mesh=plsc.VectorSubcoreMesh(core_axis_name="c", subcore_axis_name="s",
                                    num_cores=_NC, num_subcores=_NS),
        compiler_params=pltpu.CompilerParams(needs_layout_passes=False, use_tc_tiling_on_sc=False),
        scratch_types=[
            pltpu.VMEM((epw + _L,), _i32),
            pltpu.VMEM((epw + _L,), _i32),
            pltpu.VMEM((_N0 * 3,), _f32),
            pltpu.VMEM((_L,), _f32),
        ],
    )(srce, dste, pos)


_CH = 512
_CPW = 10


_MW = 48


def _edgek1_body(srce, dste, pos, x, w1, mx, agg_o,
                 src_v, dst_v, posx, xv, w1v, mxv,
                 msgbuf, dstbuf, zrows, agg_sh):
    w = _wid()
    sid = lax.axis_index("s")
    cid = _cid()
    epw = _E0 // _NW
    cap = _CH * _CPW
    base = w * epw
    zeros16 = jnp.zeros((_L,), _f32)
    zi16 = jnp.zeros((_L,), _i32)
    for t in range((cap - (epw - _L) + _L - 1) // _L):
        src_v[pl.ds(epw - _L + t * _L, _L)] = zi16
        dst_v[pl.ds(epw - _L + t * _L, _L)] = zi16
    pltpu.sync_copy(srce.at[pl.ds(base, epw)], src_v.at[pl.ds(0, epw)])
    pltpu.sync_copy(dste.at[pl.ds(base, epw)], dst_v.at[pl.ds(0, epw)])
    pltpu.sync_copy(pos, posx)
    pltpu.sync_copy(x, xv)
    pltpu.sync_copy(w1, w1v)
    pltpu.sync_copy(mx, mxv)

    def zbody(t, _):
        for cc in range(_MW // _L):
            zrows[t, pl.ds(cc * _L, _L)] = zeros16
        return 0

    lax.fori_loop(0, 125, zbody, 0)
    nslice = _N0 // _NS
    for t in range(nslice // 125):
        pltpu.sync_copy(zrows, agg_sh.at[pl.ds(sid * nslice + t * 125, 125)])
    plsc.subcore_barrier()

    mall = mxv[0, :]
    for r in range(1, _NW):
        mall = jnp.maximum(mall, mxv[r, :])
    m = lax.reduce_max(mall, (0,))
    scale = 2.0 / jnp.maximum(jnp.full((_L,), m, _f32), 1e-8)
    hi = jnp.float32(4.0 - 4e-6)
    lane0 = (lax.iota(_i32, _L) == 0).astype(_f32)

    def echunk(c, _):
        def eblock(j, _):
            e0 = c * _CH + j * _L
            s16 = src_v[pl.ds(e0, _L)]
            d16 = dst_v[pl.ds(e0, _L)]
            eids = jnp.full((_L,), e0, _i32) + lax.iota(_i32, _L)
            vmask = jnp.where(eids < epw, 1.0, 0.0).astype(_f32)
            li = []
            fr = []
            s3 = s16 * 3
            d3 = d16 * 3
            for cdim in range(3):
                col = jnp.full((_L,), cdim, _i32)
                ps = plsc.load_gather(posx, [s3 + col])
                pd = plsc.load_gather(posx, [d3 + col])
                q = jnp.clip((pd - ps) * scale + 2.0, 0.0, hi)
                l_ = jnp.minimum(q.astype(_i32), 3)
                li.append(l_)
                fr.append(q - l_.astype(_f32))
            bse = ((li[0] * _KS + li[1]) * _KS + li[2]) * 32
            xs = plsc.load_gather(xv, [s16])
            xs = xs * vmask
            dstbuf[pl.ds(j * _L, _L)] = d16
            for k in range(_L):
                bk = bse[k]
                f0, f1, f2 = fr[0][k], fr[1][k], fr[2][k]
                g0, g1, g2 = 1.0 - f0, 1.0 - f1, 1.0 - f2
                xk = xs[k]
                acc_a = jnp.zeros((_L,), _f32)
                acc_b = jnp.zeros((_L,), _f32)
                for o in range(8):
                    o0, o1, o2 = _OFFS[o]
                    wgt = (f0 if o0 else g0) * (f1 if o1 else g1) * (f2 if o2 else g2)
                    off = bk + _DOFF[o] * 32
                    acc_a = acc_a + wgt * w1v[pl.ds(off, _L)]
                    acc_b = acc_b + wgt * w1v[pl.ds(off + _L, _L)]
                erow = j * _L + k
                msgbuf[erow, pl.ds(0, _L)] = xk * acc_a
                msgbuf[erow, pl.ds(_L, _L)] = xk * acc_b
                msgbuf[erow, pl.ds(2 * _L, _L)] = vmask[k] * lane0
            return 0

        lax.fori_loop(0, _CH // _L, eblock, 0)
        pltpu.sync_copy(msgbuf, agg_sh.at[dstbuf], add=True)
        return 0

    lax.fori_loop(0, _CPW, echunk, 0)
    plsc.subcore_barrier()

    pltpu.sync_copy(agg_sh.at[pl.ds(sid * nslice, nslice)],
                    agg_o.at[cid, pl.ds(sid * nslice, nslice)])


def _edgek1(srce, dste, pos, x, w1flat, mx):
    cap = _CH * _CPW
    return pl.kernel(
        _edgek1_body,
        out_type=jax.ShapeDtypeStruct((_NC, _N0, _MW), _f32),
        mesh=plsc.VectorSubcoreMesh(core_axis_name="c", subcore_axis_name="s",
                                    num_cores=_NC, num_subcores=_NS),
        compiler_params=pltpu.CompilerParams(needs_layout_passes=False, use_tc_tiling_on_sc=False),
        scratch_types=[
            pltpu.VMEM((cap + _L,), _i32),
            pltpu.VMEM((cap + _L,), _i32),
            pltpu.VMEM((_N0 * 3,), _f32),
            pltpu.VMEM((_N0,), _f32),
            pltpu.VMEM((125 * 32,), _f32),
            pltpu.VMEM((_NW, _L), _f32),
            pltpu.VMEM((_CH, _MW), _f32),
            pltpu.VMEM((_CH,), _i32),
            pltpu.VMEM((125, _MW), _f32),
            pltpu.VMEM_SHARED((_N0, _MW), _f32),
        ],
    )(srce, dste, pos, x, w1flat, mx)


def _tc1_body(aggp_ref, x_ref, root_ref, b_ref, out_ref):
    p = aggp_ref[0] + aggp_ref[1]
    agg = p[:, :32]
    deg = p[:, 32]
    agg = agg / jnp.clip(deg, 1.0, None)[:, None]
    z = agg + jnp.dot(x_ref[...], root_ref[...],
                      preferred_element_type=_f32) + b_ref[...]
    out_ref[...] = jnp.where(z > 0, z, jnp.exp(z) - 1.0)


def _tc1(aggp, x, root, b):
    return pl.pallas_call(
        _tc1_body,
        out_shape=jax.ShapeDtypeStruct((_N0, 32), _f32),
    )(aggp, x, root, b.reshape(1, -1))


def _pseudo(pos, ei):
    src, dst = ei[0], ei[1]
    delta = pos[dst] - pos[src]
    m = jnp.max(jnp.abs(delta))
    p = delta / (2.0 * jnp.maximum(m, 1e-8)) + 0.5
    return jnp.clip(p, 0.0, 1.0 - 1e-6)


def _sconv(x, ei, pseudo, W, root, bias):
    n = x.shape[0]
    src, dst = ei[0], ei[1]
    p = pseudo * (_KS - 1)
    lo = jnp.clip(jnp.floor(p), 0, _KS - 2).astype(jnp.int32)
    frac = p - lo.astype(p.dtype)
    off = jnp.asarray(_OFFS)
    idx3 = lo[:, None, :] + off[None, :, :]
    basis = jnp.prod(jnp.where(off[None] == 1, frac[:, None, :],
                               1.0 - frac[:, None, :]), axis=-1)
    wi = (idx3[..., 0] * _KS + idx3[..., 1]) * _KS + idx3[..., 2]
    proj = jnp.einsum('ni,kio->nko', x, W)
    gath = proj[src[:, None], wi]
    msg = jnp.sum(basis[..., None] * gath, axis=1)
    agg = jnp.zeros((n, W.shape[-1]), x.dtype).at[dst].add(msg)
    deg = jnp.zeros((n,), x.dtype).at[dst].add(1.0)
    agg = agg / jnp.clip(deg, 1.0)[:, None]
    return agg + x @ root + bias


def _pool(x, pos, cluster, num_clusters):
    nx = jax.ops.segment_max(x, cluster, num_segments=num_clusters)
    nx = jnp.where(jnp.isfinite(nx), nx, 0.0)
    cnt = jax.ops.segment_sum(jnp.ones(cluster.shape, pos.dtype), cluster,
                              num_segments=num_clusters)
    npos = jax.ops.segment_sum(pos, cluster, num_segments=num_clusters)
    npos = npos / jnp.clip(cnt, 1.0)[:, None]
    return nx, npos


def _head_body(h_ref, w1_ref, b1_ref, w2_ref, b2_ref, out_ref):
    z = jnp.dot(h_ref[...], w1_ref[...], preferred_element_type=_f32)
    z = z + b1_ref[...]
    z = jnp.where(z > 0, z, jnp.exp(z) - 1.0)
    z = jnp.dot(z, w2_ref[...], preferred_element_type=_f32)
    z = z + b2_ref[...]
    m = jnp.max(z, axis=-1, keepdims=True)
    s = z - m
    lse = jnp.log(jnp.sum(jnp.exp(s), axis=-1, keepdims=True))
    out_ref[...] = s - lse


def _head(h, fc1_w, fc1_b, fc2_w, fc2_b):
    return pl.pallas_call(
        _head_body,
        out_shape=jax.ShapeDtypeStruct((1, 10), _f32),
    )(h, fc1_w, fc1_b.reshape(1, -1), fc2_w, fc2_b.reshape(1, -1))


def kernel(x, pos, edge_index, cluster1, cluster2, cluster3, cluster4,
           W1, root1, b1, W2, root2, b2, W3, root3, b3, W4, root4, b4,
           fc1_w, fc1_b, fc2_w, fc2_b):
    E = edge_index.shape[1]
    srce, dste = edge_index[0], edge_index[1]
    posf = pos.reshape(-1)
    xf = x.reshape(-1)
    mx = _maxk1(srce, dste, posf)
    w1flat = W1.reshape(-1)
    aggp = _edgek1(srce, dste, posf, xf, w1flat, mx)
    h = _tc1(aggp, x, root1, b1)
    h, pos1 = _pool(h, pos, cluster1, _N1)
    ei2 = cluster1[edge_index][:, : E // 4]
    h = jax.nn.elu(_sconv(h, ei2, _pseudo(pos1, ei2), W2, root2, b2))
    h, pos2 = _pool(h, pos1, cluster2, _N2)
    ei3 = cluster2[ei2][:, : E // 16]
    h = jax.nn.elu(_sconv(h, ei3, _pseudo(pos2, ei3), W3, root3, b3))
    h, pos3 = _pool(h, pos2, cluster3, _N3)
    ei4 = cluster3[ei3][:, : E // 64]
    h = jax.nn.elu(_sconv(h, ei4, _pseudo(pos3, ei4), W4, root4, b4))
    h = jax.ops.segment_max(h, cluster4, num_segments=_N4)
    h = jnp.where(jnp.isfinite(h), h, 0.0)
    z = h.reshape(1, -1)
    return _head(z, fc1_w, fc1_b, fc2_w, fc2_b)


# speedup vs baseline: 8.8107x; 4.0639x over previous
import jax
import jax.numpy as jnp
import numpy as np
from jax import lax
from jax.experimental import pallas as pl
from jax.experimental.pallas import tpu as pltpu
from jax.experimental.pallas import tpu_sc as plsc

_KS = 5
_N0 = 10000
_E0 = 160000
_N1, _N2, _N3, _N4 = 2500, 625, 156, 8

_NC, _NS, _L = 2, 16, 16
_NW = _NC * _NS

_OFFS = np.array([[i, j, k] for i in (0, 1) for j in (0, 1) for k in (0, 1)],
                 dtype=np.int32)
_DOFF = [(o[0] * _KS + o[1]) * _KS + o[2] for o in _OFFS]

_f32 = jnp.float32
_i32 = jnp.int32
_NEG = -3.0e38

_SCPARAMS = pltpu.CompilerParams(needs_layout_passes=False,
                                 use_tc_tiling_on_sc=False)


def _mesh():
    return plsc.VectorSubcoreMesh(core_axis_name="c", subcore_axis_name="s",
                                  num_cores=_NC, num_subcores=_NS)


def _wid():
    return lax.axis_index("s") * _NC + lax.axis_index("c")


def _iota():
    return lax.iota(_i32, _L)


def _rup(v, m):
    return -(-v // m) * m


def _maxk1_body(srce, dste, pos, out, src_v, dst_v, posx, v16):
    w = _wid()
    epw = _E0 // _NW
    base = w * epw
    zi16 = jnp.zeros((_L,), _i32)
    src_v[pl.ds(epw - _L, _L)] = zi16
    src_v[pl.ds(epw, _L)] = zi16
    dst_v[pl.ds(epw - _L, _L)] = zi16
    dst_v[pl.ds(epw, _L)] = zi16
    pltpu.sync_copy(srce.at[pl.ds(base, epw)], src_v.at[pl.ds(0, epw)])
    pltpu.sync_copy(dste.at[pl.ds(base, epw)], dst_v.at[pl.ds(0, epw)])
    pltpu.sync_copy(pos, posx)
    n_it = (epw + _L - 1) // _L

    def body(i, m):
        s4 = src_v[pl.ds(i * _L, _L)] * 4
        d4 = dst_v[pl.ds(i * _L, _L)] * 4
        acc = m
        for c in range(3):
            col = jnp.full((_L,), c, _i32)
            ps = plsc.load_gather(posx, [s4 + col])
            pd = plsc.load_gather(posx, [d4 + col])
            acc = jnp.maximum(acc, jnp.abs(pd - ps))
        return acc

    m = lax.fori_loop(0, n_it, body, jnp.zeros((_L,), _f32))
    v16[...] = m
    pltpu.sync_copy(v16, out.at[w])


def _maxk1(srce, dste, pos4f):
    epw = _E0 // _NW
    return pl.kernel(
        _maxk1_body,
        out_type=jax.ShapeDtypeStruct((_NW, _L), _f32),
        mesh=_mesh(),
        compiler_params=_SCPARAMS,
        scratch_types=[
            pltpu.VMEM((epw + _L,), _i32),
            pltpu.VMEM((epw + _L,), _i32),
            pltpu.VMEM((pos4f.shape[0],), _f32),
            pltpu.VMEM((_L,), _f32),
        ],
    )(srce, dste, pos4f)


_CH = 512
_CPW = 10
_MW1 = 48


def _edgek1_body(srce, dste, pos, x, w1, mx, agg_o,
                 src_v, dst_v, posx, xv, w1v, mxv,
                 msgbuf, dstbuf, zrows, agg_sh):
    w = _wid()
    sid = lax.axis_index("s")
    cid = lax.axis_index("c")
    epw = _E0 // _NW
    cap = _CH * _CPW
    zeros16 = jnp.zeros((_L,), _f32)
    zi16 = jnp.zeros((_L,), _i32)
    base = w * epw
    for t in range((cap - (epw - _L) + _L - 1) // _L):
        src_v[pl.ds(epw - _L + t * _L, _L)] = zi16
        dst_v[pl.ds(epw - _L + t * _L, _L)] = zi16
    pltpu.sync_copy(srce.at[pl.ds(base, epw)], src_v.at[pl.ds(0, epw)])
    pltpu.sync_copy(dste.at[pl.ds(base, epw)], dst_v.at[pl.ds(0, epw)])
    pltpu.sync_copy(pos, posx)
    pltpu.sync_copy(x, xv)
    pltpu.sync_copy(w1, w1v)
    pltpu.sync_copy(mx, mxv)

    def zbody(t, _):
        for cc in range(_MW1 // _L):
            zrows[t, pl.ds(cc * _L, _L)] = zeros16
        return 0

    lax.fori_loop(0, 125, zbody, 0)
    nslice = _N0 // _NS
    for t in range(nslice // 125):
        pltpu.sync_copy(zrows, agg_sh.at[pl.ds(sid * nslice + t * 125, 125)])
    plsc.subcore_barrier()

    mall = mxv[0, :]
    for r in range(1, _NW):
        mall = jnp.maximum(mall, mxv[r, :])
    m = lax.reduce_max(mall, (0,))
    scale = 2.0 / jnp.maximum(jnp.full((_L,), m, _f32), 1e-8)
    hi = jnp.float32(4.0 - 4e-6)
    lane0 = (_iota() == 0).astype(_f32)

    def echunk(c, _):
        def eblock(j, _):
            e0 = c * _CH + j * _L
            s16 = src_v[pl.ds(e0, _L)]
            d16 = dst_v[pl.ds(e0, _L)]
            eids = jnp.full((_L,), e0, _i32) + _iota()
            vmask = jnp.where(eids < epw, 1.0, 0.0).astype(_f32)
            li = []
            fr = []
            s4 = s16 * 4
            d4 = d16 * 4
            for cdim in range(3):
                col = jnp.full((_L,), cdim, _i32)
                ps = plsc.load_gather(posx, [s4 + col])
                pd = plsc.load_gather(posx, [d4 + col])
                q = jnp.clip((pd - ps) * scale + 2.0, 0.0, hi)
                l_ = jnp.minimum(q.astype(_i32), 3)
                li.append(l_)
                fr.append(q - l_.astype(_f32))
            bse = ((li[0] * _KS + li[1]) * _KS + li[2]) * 32
            xs = plsc.load_gather(xv, [s16]) * vmask
            dstbuf[pl.ds(j * _L, _L)] = d16
            for k in range(_L):
                bk = bse[k]
                f0, f1, f2 = fr[0][k], fr[1][k], fr[2][k]
                g0, g1, g2 = 1.0 - f0, 1.0 - f1, 1.0 - f2
                xk = xs[k]
                acc_a = jnp.zeros((_L,), _f32)
                acc_b = jnp.zeros((_L,), _f32)
                for o in range(8):
                    o0, o1, o2 = _OFFS[o]
                    wgt = (f0 if o0 else g0) * (f1 if o1 else g1) * (f2 if o2 else g2)
                    off = bk + _DOFF[o] * 32
                    acc_a = acc_a + wgt * w1v[pl.ds(off, _L)]
                    acc_b = acc_b + wgt * w1v[pl.ds(off + _L, _L)]
                erow = j * _L + k
                msgbuf[erow, pl.ds(0, _L)] = xk * acc_a
                msgbuf[erow, pl.ds(_L, _L)] = xk * acc_b
                msgbuf[erow, pl.ds(2 * _L, _L)] = vmask[k] * lane0
            return 0

        lax.fori_loop(0, _CH // _L, eblock, 0)
        pltpu.sync_copy(msgbuf, agg_sh.at[dstbuf], add=True)
        return 0

    lax.fori_loop(0, _CPW, echunk, 0)
    plsc.subcore_barrier()
    pltpu.sync_copy(agg_sh.at[pl.ds(sid * nslice, nslice)],
                    agg_o.at[cid, pl.ds(sid * nslice, nslice)])


def _edgek1(srce, dste, pos4f, xf, w1flat, mx):
    cap = _CH * _CPW
    return pl.kernel(
        _edgek1_body,
        out_type=jax.ShapeDtypeStruct((_NC, _N0, _MW1), _f32),
        mesh=_mesh(),
        compiler_params=_SCPARAMS,
        scratch_types=[
            pltpu.VMEM((cap + _L,), _i32),
            pltpu.VMEM((cap + _L,), _i32),
            pltpu.VMEM((pos4f.shape[0],), _f32),
            pltpu.VMEM((_N0,), _f32),
            pltpu.VMEM((125 * 32,), _f32),
            pltpu.VMEM((_NW, _L), _f32),
            pltpu.VMEM((_CH, _MW1), _f32),
            pltpu.VMEM((_CH,), _i32),
            pltpu.VMEM((125, _MW1), _f32),
            pltpu.VMEM_SHARED((_N0, _MW1), _f32),
        ],
    )(srce, dste, pos4f, xf, w1flat, mx)


def _tc1_body(aggp_ref, x_ref, root_ref, b_ref, out_ref):
    p = aggp_ref[0] + aggp_ref[1]
    agg = p[:, :32]
    deg = p[:, 32]
    agg = agg / jnp.clip(deg, 1.0, None)[:, None]
    z = agg + jnp.dot(x_ref[...], root_ref[...],
                      preferred_element_type=_f32) + b_ref[...]
    h = jnp.where(z > 0, z, jnp.exp(z) - 1.0)
    out_ref[...] = jnp.concatenate(
        [h, jnp.zeros((out_ref.shape[0] - _N0, 32), _f32)], axis=0)


def _tc1(aggp, x, root, b, npad):
    return pl.pallas_call(
        _tc1_body,
        out_shape=jax.ShapeDtypeStruct((npad, 32), _f32),
    )(aggp, x, root, b.reshape(1, -1))


def _make_poolk(NP, C, F, with_pos):
    CF = C * F
    CFP = _rup(CF, 256)
    MAA = max((C + 1) * F, CFP)
    PAD4 = _rup((C + 1) * 4, 256)
    SW4 = PAD4 // _NS
    RPW = NP // _NW
    RPT = NP // _NS
    BLK_A = -(-RPW // _L)
    BLK_C = -(-RPT // _L)
    HSL = BLK_A * _L * F

    def inner(hf, clp, posf, hmax_o, posq_o, clv, hsl, maxarr, posarr,
              possl, tmpb, accb, ppart_sh):
        w = _wid()
        sid = lax.axis_index("s")
        cid = lax.axis_index("c")
        zeros16 = jnp.zeros((_L,), _f32)
        negv = jnp.full((_L,), _NEG, _f32)
        cpad = jnp.full((_L,), C, _i32)
        pltpu.sync_copy(clp, clv.at[pl.ds(0, NP)])
        for t in range(2):
            clv[pl.ds(NP + t * _L, _L)] = cpad
        pltpu.sync_copy(hf.at[pl.ds(w * RPW * F, RPW * F)],
                        hsl.at[pl.ds(0, RPW * F)])

        def initm(t, _):
            maxarr[pl.ds(t * _L, _L)] = negv
            return 0

        lax.fori_loop(0, MAA // _L, initm, 0)
        if with_pos:
            pltpu.sync_copy(posf.at[pl.ds(sid * RPT * 4, RPT * 4)],
                            possl.at[pl.ds(0, RPT * 4)])

            def initp(t, _):
                posarr[pl.ds(t * _L, _L)] = zeros16
                return 0

            lax.fori_loop(0, (PAD4 + _L) // _L, initp, 0)

        def ablk(b, _):
            cl16 = clv[pl.ds(w * RPW + b * _L, _L)]
            valid = (b * _L + _iota()) < RPW
            cl16 = jnp.where(valid, cl16, cpad)
            for k in range(_L):
                bse = cl16[k] * F
                roff = (b * _L + k) * F
                for fc in range(F // _L):
                    v = hsl[pl.ds(roff + fc * _L, _L)]
                    old = maxarr[pl.ds(bse + fc * _L, _L)]
                    maxarr[pl.ds(bse + fc * _L, _L)] = jnp.maximum(old, v)
            return 0

        lax.fori_loop(0, BLK_A, ablk, 0)

        if with_pos:
            lane3 = (_iota() == 3).astype(_f32)
            ltm = (_iota() < 3).astype(_f32)

            def cblk(b, _):
                cl16 = clv[pl.ds(sid * RPT + b * _L, _L)]
                valid = (b * _L + _iota()) < RPT
                cl16 = jnp.where(valid, cl16, cpad)
                for k in range(_L):
                    pk = possl[pl.ds((b * _L + k) * 4, _L)]
                    vec = pk * ltm + lane3
                    a4 = cl16[k] * 4
                    old = posarr[pl.ds(a4, _L)]
                    posarr[pl.ds(a4, _L)] = old + vec
                return 0

            lax.fori_loop(0, BLK_C, cblk, 0)

        pltpu.sync_copy(maxarr.at[pl.ds(0, CFP)], hmax_o.at[w])
        if with_pos:
            pltpu.sync_copy(posarr.at[pl.ds(0, PAD4)], ppart_sh.at[sid])
        plsc.subcore_barrier()

        if with_pos:
            s4 = sid * SW4
            pltpu.sync_copy(ppart_sh.at[0, pl.ds(s4, SW4)], accb.at[pl.ds(0, SW4)])
            for p in range(1, _NS):
                pltpu.sync_copy(ppart_sh.at[p, pl.ds(s4, SW4)],
                                tmpb.at[pl.ds(0, SW4)])

                def pred(t, _):
                    a = accb[pl.ds(t * _L, _L)]
                    v = tmpb[pl.ds(t * _L, _L)]
                    accb[pl.ds(t * _L, _L)] = a + v
                    return 0

                lax.fori_loop(0, SW4 // _L, pred, 0)
            cix = ((_iota() >> 2) << 2) + 3

            def pdiv(t, _):
                a = accb[pl.ds(t * _L, _L)]
                cnt = plsc.load_gather(
                    accb, [jnp.full((_L,), t * _L, _i32) + cix])
                accb[pl.ds(t * _L, _L)] = a / jnp.maximum(cnt, 1.0)
                return 0

            lax.fori_loop(0, SW4 // _L, pdiv, 0)

            @pl.when(cid == 0)
            def _():
                pltpu.sync_copy(accb.at[pl.ds(0, SW4)],
                                posq_o.at[pl.ds(s4, SW4)])

    scr = [
        pltpu.VMEM((NP + 2 * _L,), _i32),
        pltpu.VMEM((HSL,), _f32),
        pltpu.VMEM((MAA,), _f32),
        pltpu.VMEM((PAD4 + _L,), _f32),
        pltpu.VMEM((BLK_C * _L * 4 + _L,), _f32),
        pltpu.VMEM((SW4,), _f32),
        pltpu.VMEM((SW4,), _f32),
        pltpu.VMEM_SHARED((_NS, PAD4), _f32),
    ]

    if with_pos:
        def body(hf, clp, posf, hmax_o, posq_o, *s):
            inner(hf, clp, posf, hmax_o, posq_o, *s)

        outs = (jax.ShapeDtypeStruct((_NW, CFP), _f32),
                jax.ShapeDtypeStruct((PAD4,), _f32))

        def callk(hf, clp, posf):
            return pl.kernel(body, out_type=outs, mesh=_mesh(),
                             compiler_params=_SCPARAMS,
                             scratch_types=scr)(hf, clp, posf)
    else:
        def body(hf, clp, hmax_o, *s):
            inner(hf, clp, None, hmax_o, None, *s)

        outs = jax.ShapeDtypeStruct((_NW, CFP), _f32)

        def callk(hf, clp, posf):
            del posf
            return pl.kernel(body, out_type=outs, mesh=_mesh(),
                             compiler_params=_SCPARAMS,
                             scratch_types=scr)(hf, clp)

    return callk, CFP, PAD4


def _make_edgek(E, CPREV, CPAD, COUT, NCH, clsizes, pad4prev):
    MW = COUT + _L
    EP1 = _rup(-(-E // _NS), _L)
    EP2 = _rup(-(-E // _NW), _L)
    NB1 = EP1 // _L
    NB2 = EP2 // _L
    RT = CPAD // _NS
    CLTOT = sum(clsizes)
    NROWS = CPREV * 125

    def body(srce, dste, cl1, cl2, cl3, posq, proj, agg_o,
             s1v, d1v, s2v, d2v, clvs, posx, idxb, rowsb, fbuf, dbuf,
             msgb, dstb, zrows, maxv16, tmpm, sem0, sem1, agg_sh, max_sh):
        w = _wid()
        sid = lax.axis_index("s")
        cid = lax.axis_index("c")
        zeros16 = jnp.zeros((_L,), _f32)
        clrefs = [cl1, cl2, cl3][:NCH]
        clofs = []
        off = 0
        for i, cr in enumerate(clrefs):
            pltpu.sync_copy(cr, clvs.at[pl.ds(off, clsizes[i])])
            clofs.append(off)
            off += clsizes[i]
        pltpu.sync_copy(posq, posx)
        pltpu.sync_copy(srce.at[pl.ds(sid * EP1, EP1)], s1v)
        pltpu.sync_copy(dste.at[pl.ds(sid * EP1, EP1)], d1v)
        pltpu.sync_copy(srce.at[pl.ds(w * EP2, EP2)], s2v)
        pltpu.sync_copy(dste.at[pl.ds(w * EP2, EP2)], d2v)

        def zbody(t, _):
            for cc in range(MW // _L):
                zrows[t, pl.ds(cc * _L, _L)] = zeros16
            return 0

        lax.fori_loop(0, RT, zbody, 0)
        pltpu.sync_copy(zrows, agg_sh.at[pl.ds(sid * RT, RT)])

        def translate(v16):
            for i in range(NCH):
                base = jnp.full((_L,), clofs[i], _i32)
                v16 = plsc.load_gather(clvs, [base + v16])
            return v16

        def mblk(b, m):
            e0 = sid * EP1 + b * _L
            s16 = translate(s1v[pl.ds(b * _L, _L)])
            d16 = translate(d1v[pl.ds(b * _L, _L)])
            eids = jnp.full((_L,), e0, _i32) + _iota()
            vm = jnp.where(eids < E, 1.0, 0.0).astype(_f32)
            s4 = s16 * 4
            d4 = d16 * 4
            for c in range(3):
                col = jnp.full((_L,), c, _i32)
                ps = plsc.load_gather(posx, [s4 + col])
                pd = plsc.load_gather(posx, [d4 + col])
                m = jnp.maximum(m, jnp.abs(pd - ps) * vm)
            return m

        m16 = lax.fori_loop(0, NB1, mblk, jnp.zeros((_L,), _f32))
        maxv16[...] = m16
        pltpu.sync_copy(maxv16, max_sh.at[sid])
        plsc.subcore_barrier()
        pltpu.sync_copy(max_sh, tmpm)
        mall = tmpm[0, :]
        for p in range(1, _NS):
            mall = jnp.maximum(mall, tmpm[p, :])
        m = lax.reduce_max(mall, (0,))
        scale = 2.0 / jnp.maximum(jnp.full((_L,), m, _f32), 1e-8)
        hi = jnp.float32(4.0 - 4e-6)
        lane0 = (_iota() == 0).astype(_f32)

        def prep(b):
            slot = b % 2
            s16 = translate(s2v[pl.ds(b * _L, _L)])
            d16 = translate(d2v[pl.ds(b * _L, _L)])
            s16 = jnp.minimum(s16, CPREV - 1)
            e0 = w * EP2 + b * _L
            eids = jnp.full((_L,), e0, _i32) + _iota()
            vm = jnp.where(eids < E, 1.0, 0.0).astype(_f32)
            li = []
            fr = []
            s4 = s16 * 4
            d4 = d16 * 4
            for c in range(3):
                col = jnp.full((_L,), c, _i32)
                ps = plsc.load_gather(posx, [s4 + col])
                pd = plsc.load_gather(posx, [d4 + col])
                q = jnp.clip((pd - ps) * scale + 2.0, 0.0, hi)
                l_ = jnp.minimum(q.astype(_i32), 3)
                li.append(l_)
                fr.append(q - l_.astype(_f32))
            cell = (li[0] * _KS + li[1]) * _KS + li[2]
            rbase = s16 * 125 + cell
            for o in range(8):
                idxb[slot, pl.ds(o * _L, _L)] = rbase + _DOFF[o]
            for c in range(3):
                fbuf[slot, c, pl.ds(0, _L)] = fr[c]
            fbuf[slot, 3, pl.ds(0, _L)] = vm
            dbuf[slot, pl.ds(0, _L)] = d16

        def fire(b):
            slot = b % 2

            @pl.when(slot == 0)
            def _():
                pltpu.async_copy(proj.at[idxb.at[0]], rowsb.at[0], sem0)

            @pl.when(slot == 1)
            def _():
                pltpu.async_copy(proj.at[idxb.at[1]], rowsb.at[1], sem1)

        def wait(b):
            slot = b % 2

            @pl.when(slot == 0)
            def _():
                pltpu.make_async_copy(proj.at[idxb.at[0]], rowsb.at[0],
                                      sem0).wait()

            @pl.when(slot == 1)
            def _():
                pltpu.make_async_copy(proj.at[idxb.at[1]], rowsb.at[1],
                                      sem1).wait()

        prep(0)
        fire(0)

        def pblk(b, _):
            slot = b % 2
            wait(b)

            @pl.when(b + 1 < NB2)
            def _():
                prep(b + 1)
                fire(b + 1)

            f0v = fbuf[slot, 0, pl.ds(0, _L)]
            f1v = fbuf[slot, 1, pl.ds(0, _L)]
            f2v = fbuf[slot, 2, pl.ds(0, _L)]
            vm = fbuf[slot, 3, pl.ds(0, _L)]
            d16 = dbuf[slot, pl.ds(0, _L)]
            g0v, g1v, g2v = 1.0 - f0v, 1.0 - f1v, 1.0 - f2v
            for k in range(_L):
                f0, f1, f2 = f0v[k], f1v[k], f2v[k]
                g0, g1, g2 = g0v[k], g1v[k], g2v[k]
                vk = vm[k]
                accs = [jnp.zeros((_L,), _f32) for _ in range(COUT // _L)]
                for o in range(8):
                    o0, o1, o2 = _OFFS[o]
                    wgt = (f0 if o0 else g0) * (f1 if o1 else g1) * (f2 if o2 else g2)
                    for cc in range(COUT // _L):
                        accs[cc] = accs[cc] + wgt * rowsb[slot, o * _L + k,
                                                         pl.ds(cc * _L, _L)]
                for cc in range(COUT // _L):
                    msgb[k, pl.ds(cc * _L, _L)] = vk * accs[cc]
                msgb[k, pl.ds(COUT, _L)] = vk * lane0
            dstb[...] = d16
            pltpu.sync_copy(msgb, agg_sh.at[dstb], add=True)
            return 0

        lax.fori_loop(0, NB2, pblk, 0)
        plsc.subcore_barrier()
        pltpu.sync_copy(agg_sh.at[pl.ds(sid * RT, RT)],
                        agg_o.at[cid, pl.ds(sid * RT, RT)])

    scr = [
        pltpu.VMEM((EP1,), _i32),
        pltpu.VMEM((EP1,), _i32),
        pltpu.VMEM((EP2,), _i32),
        pltpu.VMEM((EP2,), _i32),
        pltpu.VMEM((CLTOT,), _i32),
        pltpu.VMEM((pad4prev,), _f32),
        pltpu.VMEM((2, 8 * _L), _i32),
        pltpu.VMEM((2, 8 * _L, COUT), _f32),
        pltpu.VMEM((2, 4, _L), _f32),
        pltpu.VMEM((2, _L), _i32),
        pltpu.VMEM((_L, MW), _f32),
        pltpu.VMEM((_L,), _i32),
        pltpu.VMEM((RT, MW), _f32),
        pltpu.VMEM((_L,), _f32),
        pltpu.VMEM((_NS, _L), _f32),
        pltpu.SemaphoreType.DMA,
        pltpu.SemaphoreType.DMA,
        pltpu.VMEM_SHARED((CPAD, MW), _f32),
        pltpu.VMEM_SHARED((_NS, _L), _f32),
    ]

    def callk(srce, dste, cl1, cl2, cl3, posq, proj2d):
        return pl.kernel(
            body,
            out_type=jax.ShapeDtypeStruct((_NC, CPAD, MW), _f32),
            mesh=_mesh(),
            compiler_params=_SCPARAMS,
            scratch_types=scr,
        )(srce, dste, cl1, cl2, cl3, posq, proj2d)

    return callk


def _tca(hp_pad, Wf, root, CPAD, F, COUT, rchunk):
    W = 125 * COUT

    def body(hp_ref, w_ref, r_ref, proj_ref, rt_ref):
        hm = hp_ref[0]
        for rpart in range(1, _NW):
            hm = jnp.maximum(hm, hp_ref[rpart])
        hm = jnp.where(hm < -1e38, 0.0, hm)
        proj_ref[...] = jnp.dot(hm, w_ref[...], preferred_element_type=_f32)
        rt_ref[...] = jnp.dot(hm, r_ref[...], preferred_element_type=_f32)

    return pl.pallas_call(
        body,
        grid=(CPAD // rchunk,),
        in_specs=[
            pl.BlockSpec((_NW, rchunk, F), lambda i: (0, i, 0)),
            pl.BlockSpec((F, W), lambda i: (0, 0)),
            pl.BlockSpec((F, COUT), lambda i: (0, 0)),
        ],
        out_specs=[
            pl.BlockSpec((rchunk, W), lambda i: (i, 0)),
            pl.BlockSpec((rchunk, COUT), lambda i: (i, 0)),
        ],
        out_shape=[
            jax.ShapeDtypeStruct((CPAD, W), _f32),
            jax.ShapeDtypeStruct((CPAD, COUT), _f32),
        ],
    )(hp_pad, Wf, root)


def _tcb(aggp, rt, bias, C, COUT, NPAD):
    def body(a_ref, r_ref, b_ref, out_ref):
        p = a_ref[0] + a_ref[1]
        agg = p[:C, :COUT]
        deg = p[:C, COUT]
        z = agg / jnp.clip(deg, 1.0, None)[:, None] + r_ref[:C, :] + b_ref[...]
        h = jnp.where(z > 0, z, jnp.exp(z) - 1.0)
        out_ref[...] = jnp.concatenate(
            [h, jnp.zeros((NPAD - C, COUT), _f32)], axis=0)

    return pl.pallas_call(
        body,
        out_shape=jax.ShapeDtypeStruct((NPAD, COUT), _f32),
    )(aggp, rt, bias.reshape(1, -1))


def _head_body(h_ref, w1_ref, b1_ref, w2_ref, b2_ref, out_ref):
    hm = h_ref[0:1, :]
    for rpart in range(1, _NW):
        hm = jnp.maximum(hm, h_ref[rpart:rpart + 1, :])
    hm = jnp.where(hm < -1e38, 0.0, hm)
    z = jnp.dot(hm, w1_ref[...], preferred_element_type=_f32) + b1_ref[...]
    z = jnp.where(z > 0, z, jnp.exp(z) - 1.0)
    z = jnp.dot(z, w2_ref[...], preferred_element_type=_f32) + b2_ref[...]
    m = jnp.max(z, axis=-1, keepdims=True)
    s = z - m
    lse = jnp.log(jnp.sum(jnp.exp(s), axis=-1, keepdims=True))
    out_ref[...] = s - lse


def _head(hp4, fc1_w, fc1_b, fc2_w, fc2_b):
    return pl.pallas_call(
        _head_body,
        out_shape=jax.ShapeDtypeStruct((1, 10), _f32),
    )(hp4, fc1_w, fc1_b.reshape(1, -1), fc2_w, fc2_b.reshape(1, -1))


_NP0 = 10016
_NP1 = 2528
_NP2 = 640
_NP3 = 160

_poolk1, _CFP1, _PAD41 = _make_poolk(_NP0, _N1, 32, True)
_poolk2, _CFP2, _PAD42 = _make_poolk(_NP1, _N2, 64, True)
_poolk3, _CFP3, _PAD43 = _make_poolk(_NP2, _N3, 128, True)
_poolk4, _CFP4, _ = _make_poolk(_NP3, _N4, 256, False)

_edgek2 = _make_edgek(_E0 // 4, _N1, 2560, 64, 1, [_NP0], _PAD41)
_edgek3 = _make_edgek(_E0 // 16, _N2, 640, 128, 2, [_NP0, _NP1], _PAD42)
_edgek4 = _make_edgek(_E0 // 64, _N3, 160, 256, 3, [_NP0, _NP1, _NP2], _PAD43)


def _padded_pos4(pos, n, np_):
    p = jnp.pad(pos, ((0, np_ - n), (0, 1)))
    return p.reshape(-1)


def kernel(x, pos, edge_index, cluster1, cluster2, cluster3, cluster4,
           W1, root1, b1, W2, root2, b2, W3, root3, b3, W4, root4, b4,
           fc1_w, fc1_b, fc2_w, fc2_b):
    srce, dste = edge_index[0], edge_index[1]
    pos4f = _padded_pos4(pos, _N0, _NP0)
    xf = x.reshape(-1)
    cl1p = jnp.concatenate([cluster1, jnp.full((_NP0 - _N0,), _N1, _i32)])
    cl2p = jnp.concatenate([cluster2, jnp.full((_NP1 - _N1,), _N2, _i32)])
    cl3p = jnp.concatenate([cluster3, jnp.full((_NP2 - _N2,), _N3, _i32)])
    cl4p = jnp.concatenate([cluster4, jnp.full((_NP3 - _N3,), _N4, _i32)])
    w2f = W2.transpose(1, 0, 2).reshape(32, 125 * 64)
    w3f = W3.transpose(1, 0, 2).reshape(64, 125 * 128)
    w4f = W4.transpose(1, 0, 2).reshape(128, 125 * 256)

    mx = _maxk1(srce, dste, pos4f)
    aggp1 = _edgek1(srce, dste, pos4f, xf, W1.reshape(-1), mx)
    h1 = _tc1(aggp1, x, root1, b1, _NP0)
    hm1, posq1 = _poolk1(h1.reshape(-1), cl1p, pos4f)
    hp1 = hm1[:, : _N1 * 32].reshape(_NW, _N1, 32)

    hp1p = jnp.pad(hp1, ((0, 0), (0, 2560 - _N1), (0, 0)))
    proj2, rt2 = _tca(hp1p, w2f, root2, 2560, 32, 64, 320)
    agg2 = _edgek2(srce, dste, cl1p, cl1p, cl1p, posq1,
                   proj2.reshape(2560 * 125, 64))
    h2 = _tcb(agg2, rt2, b2, _N1, 64, _NP1)
    hm2, posq2 = _poolk2(h2.reshape(-1), cl2p, posq1)
    hp2 = hm2[:, : _N2 * 64].reshape(_NW, _N2, 64)

    hp2p = jnp.pad(hp2, ((0, 0), (0, 640 - _N2), (0, 0)))
    proj3, rt3 = _tca(hp2p, w3f, root3, 640, 64, 128, 80)
    agg3 = _edgek3(srce, dste, cl1p, cl2p, cl1p, posq2,
                   proj3.reshape(640 * 125, 128))
    h3 = _tcb(agg3, rt3, b3, _N2, 128, _NP2)
    hm3, posq3 = _poolk3(h3.reshape(-1), cl3p, posq2)
    hp3 = hm3[:, : _N3 * 128].reshape(_NW, _N3, 128)

    hp3p = jnp.pad(hp3, ((0, 0), (0, 160 - _N3), (0, 0)))
    proj4, rt4 = _tca(hp3p, w4f, root4, 160, 128, 256, 40)
    agg4 = _edgek4(srce, dste, cl1p, cl2p, cl3p, posq3,
                   proj4.reshape(160 * 125, 256))
    h4 = _tcb(agg4, rt4, b4, _N3, 256, _NP3)
    hm4 = _poolk4(h4.reshape(-1), cl4p, None)
    hp4 = hm4[:, : _N4 * 256].reshape(_NW, 2048)

    return _head(hp4, fc1_w, fc1_b, fc2_w, fc2_b)


# speedup vs baseline: 9.2881x; 1.0542x over previous
import jax
import jax.numpy as jnp
import numpy as np
from jax import lax
from jax.experimental import pallas as pl
from jax.experimental.pallas import tpu as pltpu
from jax.experimental.pallas import tpu_sc as plsc

_KS = 5
_N0 = 10000
_E0 = 160000
_N1, _N2, _N3, _N4 = 2500, 625, 156, 8

_NC, _NS, _L = 2, 16, 16
_NW = _NC * _NS

_OFFS = np.array([[i, j, k] for i in (0, 1) for j in (0, 1) for k in (0, 1)],
                 dtype=np.int32)
_DOFF = [(o[0] * _KS + o[1]) * _KS + o[2] for o in _OFFS]

_f32 = jnp.float32
_i32 = jnp.int32
_NEG = -3.0e38

_SCPARAMS = pltpu.CompilerParams(needs_layout_passes=False,
                                 use_tc_tiling_on_sc=False)


def _mesh():
    return plsc.VectorSubcoreMesh(core_axis_name="c", subcore_axis_name="s",
                                  num_cores=_NC, num_subcores=_NS)


def _wid():
    return lax.axis_index("s") * _NC + lax.axis_index("c")


def _iota():
    return lax.iota(_i32, _L)


def _rup(v, m):
    return -(-v // m) * m


def _maxk1_body(srce, dste, pos, out, src_v, dst_v, posx, v16):
    w = _wid()
    epw = _E0 // _NW
    base = w * epw
    zi16 = jnp.zeros((_L,), _i32)
    src_v[pl.ds(epw - _L, _L)] = zi16
    src_v[pl.ds(epw, _L)] = zi16
    dst_v[pl.ds(epw - _L, _L)] = zi16
    dst_v[pl.ds(epw, _L)] = zi16
    pltpu.sync_copy(srce.at[pl.ds(base, epw)], src_v.at[pl.ds(0, epw)])
    pltpu.sync_copy(dste.at[pl.ds(base, epw)], dst_v.at[pl.ds(0, epw)])
    pltpu.sync_copy(pos, posx)
    n_it = (epw + _L - 1) // _L

    def body(i, m):
        s4 = src_v[pl.ds(i * _L, _L)] * 4
        d4 = dst_v[pl.ds(i * _L, _L)] * 4
        acc = m
        for c in range(3):
            col = jnp.full((_L,), c, _i32)
            ps = plsc.load_gather(posx, [s4 + col])
            pd = plsc.load_gather(posx, [d4 + col])
            acc = jnp.maximum(acc, jnp.abs(pd - ps))
        return acc

    m = lax.fori_loop(0, n_it, body, jnp.zeros((_L,), _f32))
    v16[...] = m
    pltpu.sync_copy(v16, out.at[w])


def _maxk1(srce, dste, pos4f):
    epw = _E0 // _NW
    return pl.kernel(
        _maxk1_body,
        out_type=jax.ShapeDtypeStruct((_NW, _L), _f32),
        mesh=_mesh(),
        compiler_params=_SCPARAMS,
        scratch_types=[
            pltpu.VMEM((epw + _L,), _i32),
            pltpu.VMEM((epw + _L,), _i32),
            pltpu.VMEM((pos4f.shape[0],), _f32),
            pltpu.VMEM((_L,), _f32),
        ],
    )(srce, dste, pos4f)


_CH = 512
_CPW = 10
_MW1 = 48


def _edgek1_body(srce, dste, pos, x, w1, mx, agg_o,
                 src_v, dst_v, posx, xv, w1v, mxv,
                 msgbuf, dstbuf, zrows, agg_sh):
    w = _wid()
    sid = lax.axis_index("s")
    cid = lax.axis_index("c")
    epw = _E0 // _NW
    cap = _CH * _CPW
    zeros16 = jnp.zeros((_L,), _f32)
    zi16 = jnp.zeros((_L,), _i32)
    base = w * epw
    for t in range((cap - (epw - _L) + _L - 1) // _L):
        src_v[pl.ds(epw - _L + t * _L, _L)] = zi16
        dst_v[pl.ds(epw - _L + t * _L, _L)] = zi16
    pltpu.sync_copy(srce.at[pl.ds(base, epw)], src_v.at[pl.ds(0, epw)])
    pltpu.sync_copy(dste.at[pl.ds(base, epw)], dst_v.at[pl.ds(0, epw)])
    pltpu.sync_copy(pos, posx)
    pltpu.sync_copy(x, xv)
    pltpu.sync_copy(w1, w1v)
    pltpu.sync_copy(mx, mxv)

    def zbody(t, _):
        for cc in range(_MW1 // _L):
            zrows[t, pl.ds(cc * _L, _L)] = zeros16
        return 0

    lax.fori_loop(0, 125, zbody, 0)
    nslice = _N0 // _NS
    for t in range(nslice // 125):
        pltpu.sync_copy(zrows, agg_sh.at[pl.ds(sid * nslice + t * 125, 125)])
    plsc.subcore_barrier()

    mall = mxv[0, :]
    for r in range(1, _NW):
        mall = jnp.maximum(mall, mxv[r, :])
    m = lax.reduce_max(mall, (0,))
    scale = 2.0 / jnp.maximum(jnp.full((_L,), m, _f32), 1e-8)
    hi = jnp.float32(4.0 - 4e-6)
    lane0 = (_iota() == 0).astype(_f32)

    def echunk(c, _):
        def eblock(j, _):
            e0 = c * _CH + j * _L
            s16 = src_v[pl.ds(e0, _L)]
            d16 = dst_v[pl.ds(e0, _L)]
            eids = jnp.full((_L,), e0, _i32) + _iota()
            vmask = jnp.where(eids < epw, 1.0, 0.0).astype(_f32)
            li = []
            fr = []
            s4 = s16 * 4
            d4 = d16 * 4
            for cdim in range(3):
                col = jnp.full((_L,), cdim, _i32)
                ps = plsc.load_gather(posx, [s4 + col])
                pd = plsc.load_gather(posx, [d4 + col])
                q = jnp.clip((pd - ps) * scale + 2.0, 0.0, hi)
                l_ = jnp.minimum(q.astype(_i32), 3)
                li.append(l_)
                fr.append(q - l_.astype(_f32))
            bse = ((li[0] * _KS + li[1]) * _KS + li[2]) * 32
            xs = plsc.load_gather(xv, [s16]) * vmask
            dstbuf[pl.ds(j * _L, _L)] = d16
            for k in range(_L):
                bk = bse[k]
                f0, f1, f2 = fr[0][k], fr[1][k], fr[2][k]
                g0, g1, g2 = 1.0 - f0, 1.0 - f1, 1.0 - f2
                xk = xs[k]
                acc_a = jnp.zeros((_L,), _f32)
                acc_b = jnp.zeros((_L,), _f32)
                for o in range(8):
                    o0, o1, o2 = _OFFS[o]
                    wgt = (f0 if o0 else g0) * (f1 if o1 else g1) * (f2 if o2 else g2)
                    off = bk + _DOFF[o] * 32
                    acc_a = acc_a + wgt * w1v[pl.ds(off, _L)]
                    acc_b = acc_b + wgt * w1v[pl.ds(off + _L, _L)]
                erow = j * _L + k
                msgbuf[erow, pl.ds(0, _L)] = xk * acc_a
                msgbuf[erow, pl.ds(_L, _L)] = xk * acc_b
                msgbuf[erow, pl.ds(2 * _L, _L)] = vmask[k] * lane0
            return 0

        lax.fori_loop(0, _CH // _L, eblock, 0)
        pltpu.sync_copy(msgbuf, agg_sh.at[dstbuf], add=True)
        return 0

    lax.fori_loop(0, _CPW, echunk, 0)
    plsc.subcore_barrier()
    pltpu.sync_copy(agg_sh.at[pl.ds(sid * nslice, nslice)],
                    agg_o.at[cid, pl.ds(sid * nslice, nslice)])


def _edgek1(srce, dste, pos4f, xf, w1flat, mx):
    cap = _CH * _CPW
    return pl.kernel(
        _edgek1_body,
        out_type=jax.ShapeDtypeStruct((_NC, _N0, _MW1), _f32),
        mesh=_mesh(),
        compiler_params=_SCPARAMS,
        scratch_types=[
            pltpu.VMEM((cap + _L,), _i32),
            pltpu.VMEM((cap + _L,), _i32),
            pltpu.VMEM((pos4f.shape[0],), _f32),
            pltpu.VMEM((_N0,), _f32),
            pltpu.VMEM((125 * 32,), _f32),
            pltpu.VMEM((_NW, _L), _f32),
            pltpu.VMEM((_CH, _MW1), _f32),
            pltpu.VMEM((_CH,), _i32),
            pltpu.VMEM((125, _MW1), _f32),
            pltpu.VMEM_SHARED((_N0, _MW1), _f32),
        ],
    )(srce, dste, pos4f, xf, w1flat, mx)


def _tc1_body(aggp_ref, x_ref, root_ref, b_ref, out_ref):
    p = aggp_ref[0] + aggp_ref[1]
    agg = p[:, :32]
    deg = p[:, 32]
    agg = agg / jnp.clip(deg, 1.0, None)[:, None]
    z = agg + jnp.dot(x_ref[...], root_ref[...],
                      preferred_element_type=_f32) + b_ref[...]
    h = jnp.where(z > 0, z, jnp.exp(z) - 1.0)
    out_ref[...] = jnp.concatenate(
        [h, jnp.zeros((out_ref.shape[0] - _N0, 32), _f32)], axis=0)


def _tc1(aggp, x, root, b, npad):
    return pl.pallas_call(
        _tc1_body,
        out_shape=jax.ShapeDtypeStruct((npad, 32), _f32),
    )(aggp, x, root, b.reshape(1, -1))


def _make_poolk(NP, C, F, with_pos):
    CF = C * F
    CFP = _rup(CF, 256)
    MAA = max((C + 1) * F, CFP)
    PAD4 = _rup((C + 1) * 4, 256)
    SW4 = PAD4 // _NS
    RPW = NP // _NW
    RPT = NP // _NS
    BLK_A = -(-RPW // _L)
    BLK_C = -(-RPT // _L)
    HSL = BLK_A * _L * F

    def inner(hf, clp, posf, hmax_o, posq_o, clv, hsl, maxarr, posarr,
              possl, tmpb, accb, ppart_sh):
        w = _wid()
        sid = lax.axis_index("s")
        cid = lax.axis_index("c")
        zeros16 = jnp.zeros((_L,), _f32)
        negv = jnp.full((_L,), _NEG, _f32)
        cpad = jnp.full((_L,), C, _i32)
        pltpu.sync_copy(clp, clv.at[pl.ds(0, NP)])
        for t in range(2):
            clv[pl.ds(NP + t * _L, _L)] = cpad
        pltpu.sync_copy(hf.at[pl.ds(w * RPW * F, RPW * F)],
                        hsl.at[pl.ds(0, RPW * F)])

        def initm(t, _):
            maxarr[pl.ds(t * _L, _L)] = negv
            return 0

        lax.fori_loop(0, MAA // _L, initm, 0)
        if with_pos:
            pltpu.sync_copy(posf.at[pl.ds(sid * RPT * 4, RPT * 4)],
                            possl.at[pl.ds(0, RPT * 4)])

            def initp(t, _):
                posarr[pl.ds(t * _L, _L)] = zeros16
                return 0

            lax.fori_loop(0, (PAD4 + _L) // _L, initp, 0)

        def ablk(b, _):
            cl16 = clv[pl.ds(w * RPW + b * _L, _L)]
            valid = (b * _L + _iota()) < RPW
            cl16 = jnp.where(valid, cl16, cpad)
            for k in range(_L):
                bse = cl16[k] * F
                roff = (b * _L + k) * F
                for fc in range(F // _L):
                    v = hsl[pl.ds(roff + fc * _L, _L)]
                    old = maxarr[pl.ds(bse + fc * _L, _L)]
                    maxarr[pl.ds(bse + fc * _L, _L)] = jnp.maximum(old, v)
            return 0

        lax.fori_loop(0, BLK_A, ablk, 0)

        if with_pos:
            lane3 = (_iota() == 3).astype(_f32)
            ltm = (_iota() < 3).astype(_f32)

            def cblk(b, _):
                cl16 = clv[pl.ds(sid * RPT + b * _L, _L)]
                valid = (b * _L + _iota()) < RPT
                cl16 = jnp.where(valid, cl16, cpad)
                for k in range(_L):
                    pk = possl[pl.ds((b * _L + k) * 4, _L)]
                    vec = pk * ltm + lane3
                    a4 = cl16[k] * 4
                    old = posarr[pl.ds(a4, _L)]
                    posarr[pl.ds(a4, _L)] = old + vec
                return 0

            lax.fori_loop(0, BLK_C, cblk, 0)

        pltpu.sync_copy(maxarr.at[pl.ds(0, CFP)], hmax_o.at[w])
        if with_pos:
            pltpu.sync_copy(posarr.at[pl.ds(0, PAD4)], ppart_sh.at[sid])
        plsc.subcore_barrier()

        if with_pos:
            s4 = sid * SW4
            pltpu.sync_copy(ppart_sh.at[0, pl.ds(s4, SW4)], accb.at[pl.ds(0, SW4)])
            for p in range(1, _NS):
                pltpu.sync_copy(ppart_sh.at[p, pl.ds(s4, SW4)],
                                tmpb.at[pl.ds(0, SW4)])

                def pred(t, _):
                    a = accb[pl.ds(t * _L, _L)]
                    v = tmpb[pl.ds(t * _L, _L)]
                    accb[pl.ds(t * _L, _L)] = a + v
                    return 0

                lax.fori_loop(0, SW4 // _L, pred, 0)
            cix = ((_iota() >> 2) << 2) + 3

            def pdiv(t, _):
                a = accb[pl.ds(t * _L, _L)]
                cnt = plsc.load_gather(
                    accb, [jnp.full((_L,), t * _L, _i32) + cix])
                accb[pl.ds(t * _L, _L)] = a / jnp.maximum(cnt, 1.0)
                return 0

            lax.fori_loop(0, SW4 // _L, pdiv, 0)

            @pl.when(cid == 0)
            def _():
                pltpu.sync_copy(accb.at[pl.ds(0, SW4)],
                                posq_o.at[pl.ds(s4, SW4)])

    scr = [
        pltpu.VMEM((NP + 2 * _L,), _i32),
        pltpu.VMEM((HSL,), _f32),
        pltpu.VMEM((MAA,), _f32),
        pltpu.VMEM((PAD4 + _L,), _f32),
        pltpu.VMEM((BLK_C * _L * 4 + _L,), _f32),
        pltpu.VMEM((SW4,), _f32),
        pltpu.VMEM((SW4,), _f32),
        pltpu.VMEM_SHARED((_NS, PAD4), _f32),
    ]

    if with_pos:
        def body(hf, clp, posf, hmax_o, posq_o, *s):
            inner(hf, clp, posf, hmax_o, posq_o, *s)

        outs = (jax.ShapeDtypeStruct((_NW, CFP), _f32),
                jax.ShapeDtypeStruct((PAD4,), _f32))

        def callk(hf, clp, posf):
            return pl.kernel(body, out_type=outs, mesh=_mesh(),
                             compiler_params=_SCPARAMS,
                             scratch_types=scr)(hf, clp, posf)
    else:
        def body(hf, clp, hmax_o, *s):
            inner(hf, clp, None, hmax_o, None, *s)

        outs = jax.ShapeDtypeStruct((_NW, CFP), _f32)

        def callk(hf, clp, posf):
            del posf
            return pl.kernel(body, out_type=outs, mesh=_mesh(),
                             compiler_params=_SCPARAMS,
                             scratch_types=scr)(hf, clp)

    return callk, CFP, PAD4


def _make_edgek(E, CPREV, CPAD, COUT, NCH, clsizes, pad4prev, DEPTH=4):
    MW = COUT + _L
    EP1 = _rup(-(-E // _NS), _L)
    EP2 = _rup(-(-E // _NW), _L)
    NB1 = EP1 // _L
    NB2 = EP2 // _L
    RT = CPAD // _NS
    CLTOT = sum(clsizes)
    NROWS = CPREV * 125

    def body(srce, dste, cl1, cl2, cl3, posq, proj, agg_o,
             s1v, d1v, s2v, d2v, clvs, posx, idxb, rowsb, fbuf, dbuf,
             msgb, dstb, zrows, maxv16, tmpm, *rest):
        sems = rest[:DEPTH]
        agg_sh, max_sh = rest[DEPTH], rest[DEPTH + 1]
        w = _wid()
        sid = lax.axis_index("s")
        cid = lax.axis_index("c")
        zeros16 = jnp.zeros((_L,), _f32)
        clrefs = [cl1, cl2, cl3][:NCH]
        clofs = []
        off = 0
        for i, cr in enumerate(clrefs):
            pltpu.sync_copy(cr, clvs.at[pl.ds(off, clsizes[i])])
            clofs.append(off)
            off += clsizes[i]
        pltpu.sync_copy(posq, posx)
        pltpu.sync_copy(srce.at[pl.ds(sid * EP1, EP1)], s1v)
        pltpu.sync_copy(dste.at[pl.ds(sid * EP1, EP1)], d1v)
        pltpu.sync_copy(srce.at[pl.ds(w * EP2, EP2)], s2v)
        pltpu.sync_copy(dste.at[pl.ds(w * EP2, EP2)], d2v)

        def zbody(t, _):
            for cc in range(MW // _L):
                zrows[t, pl.ds(cc * _L, _L)] = zeros16
            return 0

        lax.fori_loop(0, RT, zbody, 0)
        pltpu.sync_copy(zrows, agg_sh.at[pl.ds(sid * RT, RT)])

        def translate(v16):
            for i in range(NCH):
                base = jnp.full((_L,), clofs[i], _i32)
                v16 = plsc.load_gather(clvs, [base + v16])
            return v16

        def mblk(b, m):
            e0 = sid * EP1 + b * _L
            s16 = translate(s1v[pl.ds(b * _L, _L)])
            d16 = translate(d1v[pl.ds(b * _L, _L)])
            eids = jnp.full((_L,), e0, _i32) + _iota()
            vm = jnp.where(eids < E, 1.0, 0.0).astype(_f32)
            s4 = s16 * 4
            d4 = d16 * 4
            for c in range(3):
                col = jnp.full((_L,), c, _i32)
                ps = plsc.load_gather(posx, [s4 + col])
                pd = plsc.load_gather(posx, [d4 + col])
                m = jnp.maximum(m, jnp.abs(pd - ps) * vm)
            return m

        m16 = lax.fori_loop(0, NB1, mblk, jnp.zeros((_L,), _f32))
        maxv16[...] = m16
        pltpu.sync_copy(maxv16, max_sh.at[sid])
        plsc.subcore_barrier()
        pltpu.sync_copy(max_sh, tmpm)
        mall = tmpm[0, :]
        for p in range(1, _NS):
            mall = jnp.maximum(mall, tmpm[p, :])
        m = lax.reduce_max(mall, (0,))
        scale = 2.0 / jnp.maximum(jnp.full((_L,), m, _f32), 1e-8)
        hi = jnp.float32(4.0 - 4e-6)
        lane0 = (_iota() == 0).astype(_f32)

        def prep(b):
            slot = b % DEPTH
            s16 = translate(s2v[pl.ds(b * _L, _L)])
            d16 = translate(d2v[pl.ds(b * _L, _L)])
            s16 = jnp.minimum(s16, CPREV - 1)
            e0 = w * EP2 + b * _L
            eids = jnp.full((_L,), e0, _i32) + _iota()
            vm = jnp.where(eids < E, 1.0, 0.0).astype(_f32)
            li = []
            fr = []
            s4 = s16 * 4
            d4 = d16 * 4
            for c in range(3):
                col = jnp.full((_L,), c, _i32)
                ps = plsc.load_gather(posx, [s4 + col])
                pd = plsc.load_gather(posx, [d4 + col])
                q = jnp.clip((pd - ps) * scale + 2.0, 0.0, hi)
                l_ = jnp.minimum(q.astype(_i32), 3)
                li.append(l_)
                fr.append(q - l_.astype(_f32))
            cell = (li[0] * _KS + li[1]) * _KS + li[2]
            rbase = s16 * 125 + cell
            for o in range(8):
                idxb[slot, pl.ds(o * _L, _L)] = rbase + _DOFF[o]
            for c in range(3):
                fbuf[slot, c, pl.ds(0, _L)] = fr[c]
            fbuf[slot, 3, pl.ds(0, _L)] = vm
            dbuf[slot, pl.ds(0, _L)] = d16

        def fire(b):
            slot = b % DEPTH
            for dd in range(DEPTH):
                @pl.when(slot == dd)
                def _(dd=dd):
                    pltpu.async_copy(proj.at[idxb.at[dd]], rowsb.at[dd],
                                     sems[dd])

        def wait(b):
            slot = b % DEPTH
            for dd in range(DEPTH):
                @pl.when(slot == dd)
                def _(dd=dd):
                    pltpu.make_async_copy(proj.at[idxb.at[dd]], rowsb.at[dd],
                                          sems[dd]).wait()

        for bb in range(min(DEPTH - 1, NB2)):
            prep(bb)
            fire(bb)

        def pblk(b, _):
            slot = b % DEPTH
            wait(b)

            @pl.when(b + DEPTH - 1 < NB2)
            def _():
                prep(b + DEPTH - 1)
                fire(b + DEPTH - 1)

            f0v = fbuf[slot, 0, pl.ds(0, _L)]
            f1v = fbuf[slot, 1, pl.ds(0, _L)]
            f2v = fbuf[slot, 2, pl.ds(0, _L)]
            vm = fbuf[slot, 3, pl.ds(0, _L)]
            d16 = dbuf[slot, pl.ds(0, _L)]
            g0v, g1v, g2v = 1.0 - f0v, 1.0 - f1v, 1.0 - f2v
            for k in range(_L):
                f0, f1, f2 = f0v[k], f1v[k], f2v[k]
                g0, g1, g2 = g0v[k], g1v[k], g2v[k]
                vk = vm[k]
                accs = [jnp.zeros((_L,), _f32) for _ in range(COUT // _L)]
                for o in range(8):
                    o0, o1, o2 = _OFFS[o]
                    wgt = (f0 if o0 else g0) * (f1 if o1 else g1) * (f2 if o2 else g2)
                    for cc in range(COUT // _L):
                        accs[cc] = accs[cc] + wgt * rowsb[slot, o * _L + k,
                                                         pl.ds(cc * _L, _L)]
                for cc in range(COUT // _L):
                    msgb[k, pl.ds(cc * _L, _L)] = vk * accs[cc]
                msgb[k, pl.ds(COUT, _L)] = vk * lane0
            dstb[...] = d16
            pltpu.sync_copy(msgb, agg_sh.at[dstb], add=True)
            return 0

        lax.fori_loop(0, NB2, pblk, 0)
        plsc.subcore_barrier()
        pltpu.sync_copy(agg_sh.at[pl.ds(sid * RT, RT)],
                        agg_o.at[cid, pl.ds(sid * RT, RT)])

    scr = [
        pltpu.VMEM((EP1,), _i32),
        pltpu.VMEM((EP1,), _i32),
        pltpu.VMEM((EP2,), _i32),
        pltpu.VMEM((EP2,), _i32),
        pltpu.VMEM((CLTOT,), _i32),
        pltpu.VMEM((pad4prev,), _f32),
        pltpu.VMEM((DEPTH, 8 * _L), _i32),
        pltpu.VMEM((DEPTH, 8 * _L, COUT), _f32),
        pltpu.VMEM((DEPTH, 4, _L), _f32),
        pltpu.VMEM((DEPTH, _L), _i32),
        pltpu.VMEM((_L, MW), _f32),
        pltpu.VMEM((_L,), _i32),
        pltpu.VMEM((RT, MW), _f32),
        pltpu.VMEM((_L,), _f32),
        pltpu.VMEM((_NS, _L), _f32),
    ] + [pltpu.SemaphoreType.DMA] * DEPTH + [
        pltpu.VMEM_SHARED((CPAD, MW), _f32),
        pltpu.VMEM_SHARED((_NS, _L), _f32),
    ]

    def callk(srce, dste, cl1, cl2, cl3, posq, proj2d):
        return pl.kernel(
            body,
            out_type=jax.ShapeDtypeStruct((_NC, CPAD, MW), _f32),
            mesh=_mesh(),
            compiler_params=_SCPARAMS,
            scratch_types=scr,
        )(srce, dste, cl1, cl2, cl3, posq, proj2d)

    return callk


def _tca(hp_pad, Wf, root, CPAD, F, COUT, rchunk):
    W = 125 * COUT

    def body(hp_ref, w_ref, r_ref, proj_ref, rt_ref):
        hm = hp_ref[0]
        for rpart in range(1, _NW):
            hm = jnp.maximum(hm, hp_ref[rpart])
        hm = jnp.where(hm < -1e38, 0.0, hm)
        proj_ref[...] = jnp.dot(hm, w_ref[...], preferred_element_type=_f32)
        rt_ref[...] = jnp.dot(hm, r_ref[...], preferred_element_type=_f32)

    return pl.pallas_call(
        body,
        grid=(CPAD // rchunk,),
        in_specs=[
            pl.BlockSpec((_NW, rchunk, F), lambda i: (0, i, 0)),
            pl.BlockSpec((F, W), lambda i: (0, 0)),
            pl.BlockSpec((F, COUT), lambda i: (0, 0)),
        ],
        out_specs=[
            pl.BlockSpec((rchunk, W), lambda i: (i, 0)),
            pl.BlockSpec((rchunk, COUT), lambda i: (i, 0)),
        ],
        out_shape=[
            jax.ShapeDtypeStruct((CPAD, W), _f32),
            jax.ShapeDtypeStruct((CPAD, COUT), _f32),
        ],
    )(hp_pad, Wf, root)


def _tcb(aggp, rt, bias, C, COUT, NPAD):
    def body(a_ref, r_ref, b_ref, out_ref):
        p = a_ref[0] + a_ref[1]
        agg = p[:C, :COUT]
        deg = p[:C, COUT]
        z = agg / jnp.clip(deg, 1.0, None)[:, None] + r_ref[:C, :] + b_ref[...]
        h = jnp.where(z > 0, z, jnp.exp(z) - 1.0)
        out_ref[...] = jnp.concatenate(
            [h, jnp.zeros((NPAD - C, COUT), _f32)], axis=0)

    return pl.pallas_call(
        body,
        out_shape=jax.ShapeDtypeStruct((NPAD, COUT), _f32),
    )(aggp, rt, bias.reshape(1, -1))


def _head_body(h_ref, w1_ref, b1_ref, w2_ref, b2_ref, out_ref):
    hm = h_ref[0:1, :]
    for rpart in range(1, _NW):
        hm = jnp.maximum(hm, h_ref[rpart:rpart + 1, :])
    hm = jnp.where(hm < -1e38, 0.0, hm)
    z = jnp.dot(hm, w1_ref[...], preferred_element_type=_f32) + b1_ref[...]
    z = jnp.where(z > 0, z, jnp.exp(z) - 1.0)
    z = jnp.dot(z, w2_ref[...], preferred_element_type=_f32) + b2_ref[...]
    m = jnp.max(z, axis=-1, keepdims=True)
    s = z - m
    lse = jnp.log(jnp.sum(jnp.exp(s), axis=-1, keepdims=True))
    out_ref[...] = s - lse


def _head(hp4, fc1_w, fc1_b, fc2_w, fc2_b):
    return pl.pallas_call(
        _head_body,
        out_shape=jax.ShapeDtypeStruct((1, 10), _f32),
    )(hp4, fc1_w, fc1_b.reshape(1, -1), fc2_w, fc2_b.reshape(1, -1))


_NP0 = 10016
_NP1 = 2528
_NP2 = 640
_NP3 = 160

_poolk1, _CFP1, _PAD41 = _make_poolk(_NP0, _N1, 32, True)
_poolk2, _CFP2, _PAD42 = _make_poolk(_NP1, _N2, 64, True)
_poolk3, _CFP3, _PAD43 = _make_poolk(_NP2, _N3, 128, True)
_poolk4, _CFP4, _ = _make_poolk(_NP3, _N4, 256, False)

_edgek2 = _make_edgek(_E0 // 4, _N1, 2560, 64, 1, [_NP0], _PAD41)
_edgek3 = _make_edgek(_E0 // 16, _N2, 640, 128, 2, [_NP0, _NP1], _PAD42)
_edgek4 = _make_edgek(_E0 // 64, _N3, 160, 256, 3, [_NP0, _NP1, _NP2], _PAD43, DEPTH=2)


def _padded_pos4(pos, n, np_):
    p = jnp.pad(pos, ((0, np_ - n), (0, 1)))
    return p.reshape(-1)


def kernel(x, pos, edge_index, cluster1, cluster2, cluster3, cluster4,
           W1, root1, b1, W2, root2, b2, W3, root3, b3, W4, root4, b4,
           fc1_w, fc1_b, fc2_w, fc2_b):
    srce, dste = edge_index[0], edge_index[1]
    pos4f = _padded_pos4(pos, _N0, _NP0)
    xf = x.reshape(-1)
    cl1p = jnp.concatenate([cluster1, jnp.full((_NP0 - _N0,), _N1, _i32)])
    cl2p = jnp.concatenate([cluster2, jnp.full((_NP1 - _N1,), _N2, _i32)])
    cl3p = jnp.concatenate([cluster3, jnp.full((_NP2 - _N2,), _N3, _i32)])
    cl4p = jnp.concatenate([cluster4, jnp.full((_NP3 - _N3,), _N4, _i32)])
    w2f = W2.transpose(1, 0, 2).reshape(32, 125 * 64)
    w3f = W3.transpose(1, 0, 2).reshape(64, 125 * 128)
    w4f = W4.transpose(1, 0, 2).reshape(128, 125 * 256)

    mx = _maxk1(srce, dste, pos4f)
    aggp1 = _edgek1(srce, dste, pos4f, xf, W1.reshape(-1), mx)
    h1 = _tc1(aggp1, x, root1, b1, _NP0)
    hm1, posq1 = _poolk1(h1.reshape(-1), cl1p, pos4f)
    hp1 = hm1[:, : _N1 * 32].reshape(_NW, _N1, 32)

    hp1p = jnp.pad(hp1, ((0, 0), (0, 2560 - _N1), (0, 0)))
    proj2, rt2 = _tca(hp1p, w2f, root2, 2560, 32, 64, 320)
    agg2 = _edgek2(srce, dste, cl1p, cl1p, cl1p, posq1,
                   proj2.reshape(2560 * 125, 64))
    h2 = _tcb(agg2, rt2, b2, _N1, 64, _NP1)
    hm2, posq2 = _poolk2(h2.reshape(-1), cl2p, posq1)
    hp2 = hm2[:, : _N2 * 64].reshape(_NW, _N2, 64)

    hp2p = jnp.pad(hp2, ((0, 0), (0, 640 - _N2), (0, 0)))
    proj3, rt3 = _tca(hp2p, w3f, root3, 640, 64, 128, 80)
    agg3 = _edgek3(srce, dste, cl1p, cl2p, cl1p, posq2,
                   proj3.reshape(640 * 125, 128))
    h3 = _tcb(agg3, rt3, b3, _N2, 128, _NP2)
    hm3, posq3 = _poolk3(h3.reshape(-1), cl3p, posq2)
    hp3 = hm3[:, : _N3 * 128].reshape(_NW, _N3, 128)

    hp3p = jnp.pad(hp3, ((0, 0), (0, 160 - _N3), (0, 0)))
    proj4, rt4 = _tca(hp3p, w4f, root4, 160, 128, 256, 40)
    agg4 = _edgek4(srce, dste, cl1p, cl2p, cl3p, posq3,
                   proj4.reshape(160 * 125, 256))
    h4 = _tcb(agg4, rt4, b4, _N3, 256, _NP3)
    hm4 = _poolk4(h4.reshape(-1), cl4p, None)
    hp4 = hm4[:, : _N4 * 256].reshape(_NW, 2048)

    return _head(hp4, fc1_w, fc1_b, fc2_w, fc2_b)
